# baseline (device time: 13484 ns/iter reference)
import jax
import jax.numpy as jnp
from jax import lax
from jax.experimental import pallas as pl
from jax.experimental.pallas import tpu as pltpu

N_DEV = 4
N_CHUNKS = 4


def kernel(x, pi):
    _, m, n = x.shape
    rows = m // N_CHUNKS

    def body(
        x_ref,
        pi_ref,
        out_ref,
        stage,
        send_buf,
        recv_buf,
        in_sems,
        out_sems,
        send_sems,
        recv_sems,
    ):
        me = lax.axis_index("i")
        dst = pi_ref[me]
        src = jnp.int32(0)
        for j in range(N_DEV):
            src = jnp.where(pi_ref[j] == me, jnp.int32(j), src)

        barrier_sem = pltpu.get_barrier_semaphore()
        pl.semaphore_signal(
            barrier_sem, inc=1, device_id=(src,),
            device_id_type=pl.DeviceIdType.MESH,
        )

        sls = [pl.ds(k * rows, rows) for k in range(N_CHUNKS)]

        in_copies = []
        for k in range(N_CHUNKS):
            c = pltpu.make_async_copy(
                x_ref.at[0, sls[k]], stage.at[sls[k]], in_sems.at[k]
            )
            c.start()
            in_copies.append(c)

        def chunk_rdma(k):
            return pltpu.make_async_remote_copy(
                src_ref=send_buf.at[sls[k]],
                dst_ref=recv_buf.at[sls[k]],
                send_sem=send_sems.at[k],
                recv_sem=recv_sems.at[k],
                device_id=(dst,),
                device_id_type=pl.DeviceIdType.MESH,
            )

        in_copies[0].wait()
        send_buf[sls[0]] = stage[sls[0]].astype(jnp.bfloat16)

        pl.semaphore_wait(barrier_sem, 1)

        rdmas = []
        for k in range(N_CHUNKS):
            r = chunk_rdma(k)
            r.start()
            rdmas.append(r)
            if k + 1 < N_CHUNKS:
                in_copies[k + 1].wait()
                send_buf[sls[k + 1]] = stage[sls[k + 1]].astype(jnp.bfloat16)

        out_copies = []
        for k in range(N_CHUNKS):
            rdmas[k].wait_recv()
            c = pltpu.make_async_copy(
                recv_buf.at[sls[k]], out_ref.at[0, sls[k]], out_sems.at[k]
            )
            c.start()
            out_copies.append(c)
        for c in out_copies:
            c.wait()
        for r in rdmas:
            r.wait_send()

    return pl.pallas_call(
        body,
        out_shape=jax.ShapeDtypeStruct((1, m, n), jnp.bfloat16),
        in_specs=[
            pl.BlockSpec(memory_space=pl.ANY),
            pl.BlockSpec(memory_space=pltpu.SMEM),
        ],
        out_specs=pl.BlockSpec(memory_space=pl.ANY),
        scratch_shapes=[
            pltpu.VMEM((m, n), jnp.float32),
            pltpu.VMEM((m, n), jnp.bfloat16),
            pltpu.VMEM((m, n), jnp.bfloat16),
            pltpu.SemaphoreType.DMA((N_CHUNKS,)),
            pltpu.SemaphoreType.DMA((N_CHUNKS,)),
            pltpu.SemaphoreType.DMA((N_CHUNKS,)),
            pltpu.SemaphoreType.DMA((N_CHUNKS,)),
        ],
        compiler_params=pltpu.CompilerParams(collective_id=0),
    )(x, pi)


# device time: 13214 ns/iter; 1.0204x vs baseline; 1.0204x over previous
import jax
import jax.numpy as jnp
from jax import lax
from jax.experimental import pallas as pl
from jax.experimental.pallas import tpu as pltpu

N_DEV = 4
N_CHUNKS = 4


def kernel(x, pi):
    _, m, n = x.shape
    rows = m // N_CHUNKS

    def body(x_ref, pi_ref, out_ref, send_buf, send_sems, recv_sems):
        me = lax.axis_index("i")
        dst = pi_ref[me]
        src = jnp.int32(0)
        for j in range(N_DEV):
            src = jnp.where(pi_ref[j] == me, jnp.int32(j), src)

        barrier_sem = pltpu.get_barrier_semaphore()
        pl.semaphore_signal(
            barrier_sem, inc=1, device_id=(src,),
            device_id_type=pl.DeviceIdType.MESH,
        )

        send_buf[pl.ds(0, rows)] = x_ref[0, pl.ds(0, rows)].astype(jnp.bfloat16)

        pl.semaphore_wait(barrier_sem, 1)

        def chunk_rdma(k):
            sl = pl.ds(k * rows, rows)
            return pltpu.make_async_remote_copy(
                src_ref=send_buf.at[sl],
                dst_ref=out_ref.at[0].at[sl],
                send_sem=send_sems.at[k],
                recv_sem=recv_sems.at[k],
                device_id=(dst,),
                device_id_type=pl.DeviceIdType.MESH,
            )

        rdmas = []
        for k in range(N_CHUNKS):
            r = chunk_rdma(k)
            r.start()
            rdmas.append(r)
            if k + 1 < N_CHUNKS:
                sl = pl.ds((k + 1) * rows, rows)
                send_buf[sl] = x_ref[0, sl].astype(jnp.bfloat16)
        for r in rdmas:
            r.wait_send()
        for r in rdmas:
            r.wait_recv()

    return pl.pallas_call(
        body,
        out_shape=jax.ShapeDtypeStruct((1, m, n), jnp.bfloat16),
        in_specs=[
            pl.BlockSpec(memory_space=pltpu.VMEM),
            pl.BlockSpec(memory_space=pltpu.SMEM),
        ],
        out_specs=pl.BlockSpec(memory_space=pltpu.VMEM),
        scratch_shapes=[
            pltpu.VMEM((m, n), jnp.bfloat16),
            pltpu.SemaphoreType.DMA((N_CHUNKS,)),
            pltpu.SemaphoreType.DMA((N_CHUNKS,)),
        ],
        compiler_params=pltpu.CompilerParams(collective_id=0),
    )(x, pi)
